# baseline (speedup 1.0000x reference)
import functools

import jax
import jax.numpy as jnp
from jax import lax
from jax.experimental import pallas as pl
from jax.experimental.pallas import tpu as pltpu
from jax.experimental.pallas import tpu_sc as plsc

N = 10000
E = 160000
D = 256
H = 128
NC = 2
NS = 16
B = 80
NBP = 128
PAD_ROWS = 240
PN = N + PAD_ROWS
EP = NS * NBP * B
G = 8
GB2 = 2 * G
NBODY = NBP // GB2
NBUF = 4
DEPTH = 3
ZB = PN // NS
CHUNK = 8
R = 1000
MAGIC = 0x5F3759DF

_mesh = plsc.VectorSubcoreMesh(core_axis_name="c", subcore_axis_name="s")


@functools.partial(
    pl.kernel,
    out_type=(
        jax.ShapeDtypeStruct((PN,), jnp.float32),
        jax.ShapeDtypeStruct((PN, H), jnp.float32),
        jax.ShapeDtypeStruct((PN, H), jnp.float32),
        jax.ShapeDtypeStruct((PN, H), jnp.float32),
        jax.ShapeDtypeStruct((PN, H), jnp.float32),
    ),
    mesh=_mesh,
    scratch_types=[
        pltpu.VMEM((2, G, B), jnp.int32),
        pltpu.VMEM((2, G, B), jnp.int32),
        pltpu.VMEM((NBUF, B, H), jnp.float32),
        pltpu.VMEM((B,), jnp.float32),
        pltpu.VMEM((ZB + 16,), jnp.float32),
        pltpu.VMEM_SHARED((PN,), jnp.float32),
        pltpu.VMEM_SHARED((PN, H), jnp.float32),
        pltpu.SemaphoreType.DMA,
        pltpu.SemaphoreType.DMA,
        pltpu.SemaphoreType.DMA,
        pltpu.SemaphoreType.DMA,
    ],
)
def _sc_kernel(x_hbm, srcr, dstr, ones_hbm,
               deg_out, outa, outb, xta, xtb,
               src_v, dst_v, rows_v, ones_v, degb, dacc, acc,
               gsem, isem, ssem, dsem):
    c = lax.axis_index("c")
    s = lax.axis_index("s")
    r0 = pl.multiple_of(s * ZB, 8)
    coff = pl.multiple_of(c * H, H)

    for i in range(ZB // 16):
        degb[pl.ds(16 * i, 16)] = jnp.zeros((16,), jnp.float32)
    pltpu.sync_copy(degb.at[pl.ds(0, ZB)], dacc.at[pl.ds(r0, ZB)])
    pltpu.sync_copy(ones_hbm, ones_v)
    plsc.subcore_barrier()

    def dst_load(slot, g):
        off = pl.multiple_of(g * G, G)
        pltpu.async_copy(dstr.at[s, pl.ds(off, G)], dst_v.at[slot], isem)

    def dst_wait(slot):
        pltpu.make_async_copy(
            dstr.at[s, pl.ds(0, G)], dst_v.at[slot], isem).wait()

    def fire8(slot):
        for b in range(G):
            pltpu.async_copy(ones_v, dacc.at[dst_v.at[slot, b]], dsem,
                             add=True)

    def drain(k):
        for _ in range(k):
            pltpu.make_async_copy(ones_v, dacc.at[pl.ds(0, B)], dsem).wait()

    pltpu.sync_copy(dstr.at[s, pl.ds(0, G)], dst_v.at[0])

    def pa_body(i, carry):
        @pl.when(i > 0)
        def _():
            drain(G)
            dst_wait(0)
        dst_load(1, 2 * i + 1)
        fire8(0)
        dst_wait(1)
        fire8(1)

        @pl.when(i < NBODY - 1)
        def _():
            drain(G)
            dst_load(0, 2 * i + 2)
        return carry

    lax.fori_loop(0, NBODY, pa_body, 0)
    drain(2 * G)
    plsc.subcore_barrier()

    pltpu.sync_copy(dacc.at[pl.ds(r0, ZB)], degb.at[pl.ds(0, ZB)])

    @pl.when(c == 0)
    def _():
        pltpu.sync_copy(degb.at[pl.ds(0, ZB)], deg_out.at[pl.ds(r0, ZB)])

    for v in range(ZB // 16):
        sl = pl.ds(16 * v, 16)
        d = degb[sl] + 1.0
        y = lax.bitcast_convert_type(
            jnp.int32(MAGIC) - lax.shift_right_logical(
                lax.bitcast_convert_type(d, jnp.int32), 1), jnp.float32)
        for _ in range(3):
            y = y * (1.5 - 0.5 * d * y * y)
        degb[sl] = y

    def run(tab, out):
        for k in range(CHUNK):
            def chunk(k=k):
                row0 = pl.multiple_of(r0 + 80 * k, 8)
                pltpu.sync_copy(
                    x_hbm.at[pl.ds(row0, 80), pl.ds(coff, H)], rows_v.at[0])

                def row_scale(r, carry):
                    dv = degb[pl.ds(80 * k + r, 16)]
                    yv = jnp.broadcast_to(dv[0], (16,))
                    for j in range(H // 16):
                        sl = pl.ds(16 * j, 16)
                        rows_v[0, r, sl] = rows_v[0, r, sl] * yv
                    return carry

                lax.fori_loop(0, 80, row_scale, 0)
                pltpu.sync_copy(rows_v.at[0], acc.at[pl.ds(row0, 80)])
                pltpu.sync_copy(rows_v.at[0], tab.at[pl.ds(row0, 80)])

            if k < 5:
                chunk()
            else:
                pl.when(s < NS - 1)(chunk)
        plsc.subcore_barrier()

        def idx_load(slot, g):
            off = pl.multiple_of(g * G, G)
            pltpu.async_copy(srcr.at[s, pl.ds(off, G)], src_v.at[slot], isem)
            pltpu.async_copy(dstr.at[s, pl.ds(off, G)], dst_v.at[slot], isem)

        def idx_wait(slot):
            pltpu.make_async_copy(
                srcr.at[s, pl.ds(0, G)], src_v.at[slot], isem).wait()
            pltpu.make_async_copy(
                dstr.at[s, pl.ds(0, G)], dst_v.at[slot], isem).wait()

        def gather(slot, row, buf):
            pltpu.async_copy(tab.at[src_v.at[slot, row]], rows_v.at[buf], gsem)

        def gather_wait(buf):
            pltpu.make_async_copy(
                tab.at[pl.ds(0, B)], rows_v.at[buf], gsem).wait()

        def scatter_wait(buf):
            pltpu.make_async_copy(
                rows_v.at[buf], acc.at[pl.ds(0, B)], ssem).wait()

        pltpu.sync_copy(srcr.at[s, pl.ds(0, G)], src_v.at[0])
        pltpu.sync_copy(dstr.at[s, pl.ds(0, G)], dst_v.at[0])
        for b in range(DEPTH):
            gather(0, b, b)

        def body(i, carry):
            idx_load(1, 2 * i + 1)
            for t in range(GB2):
                gather_wait(t % NBUF)
                if t == 0:
                    @pl.when(i > 0)
                    def _():
                        scatter_wait((t + 3) % NBUF)
                else:
                    scatter_wait((t + 3) % NBUF)
                nt = t + DEPTH
                if nt < GB2:
                    slot, row = (0, nt) if nt < G else (1, nt - G)
                    if nt == G:
                        idx_wait(1)
                    gather(slot, row, nt % NBUF)
                else:
                    @pl.when(i < NBODY - 1)
                    def _(nt=nt):
                        if nt == GB2:
                            idx_wait(0)
                        gather(0, nt - GB2, nt % NBUF)
                if t == G:
                    @pl.when(i < NBODY - 1)
                    def _():
                        idx_load(0, 2 * i + 2)
                sl, rw = (0, t) if t < G else (1, t - G)
                pltpu.async_copy(
                    rows_v.at[t % NBUF], acc.at[dst_v.at[sl, rw]], ssem,
                    add=True)
            return carry

        lax.fori_loop(0, NBODY, body, 0)
        scatter_wait((NBP - 1) % NBUF)
        plsc.subcore_barrier()
        pltpu.sync_copy(acc.at[pl.ds(r0, ZB)], out.at[pl.ds(r0, ZB)])

    @pl.when(c == 0)
    def _():
        run(xta, outa)

    @pl.when(c == 1)
    def _():
        run(xtb, outb)


def _mm_body(deg_ref, sa_ref, sb_ref, wmu_ref, wls_ref, bmu_ref, bls_ref,
             mu_ref, ls_ref):
    dinv = lax.rsqrt(deg_ref[...] + 1.0)
    p0 = sa_ref[...] * dinv
    p1 = sb_ref[...] * dinv
    wmu = wmu_ref[...]
    wls = wls_ref[...]
    mu_ref[...] = (
        jnp.dot(p0, wmu[:H, :], preferred_element_type=jnp.float32)
        + jnp.dot(p1, wmu[H:, :], preferred_element_type=jnp.float32)
        + bmu_ref[...]
    )
    ls_ref[...] = (
        jnp.dot(p0, wls[:H, :], preferred_element_type=jnp.float32)
        + jnp.dot(p1, wls[H:, :], preferred_element_type=jnp.float32)
        + bls_ref[...]
    )


_mm = pl.pallas_call(
    _mm_body,
    grid=(N // R,),
    in_specs=[
        pl.BlockSpec((R, 1), lambda i: (i, 0)),
        pl.BlockSpec((R, H), lambda i: (i, 0)),
        pl.BlockSpec((R, H), lambda i: (i, 0)),
        pl.BlockSpec((D, D), lambda i: (0, 0)),
        pl.BlockSpec((D, D), lambda i: (0, 0)),
        pl.BlockSpec((1, D), lambda i: (0, 0)),
        pl.BlockSpec((1, D), lambda i: (0, 0)),
    ],
    out_specs=[
        pl.BlockSpec((R, D), lambda i: (i, 0)),
        pl.BlockSpec((R, D), lambda i: (i, 0)),
    ],
    out_shape=[
        jax.ShapeDtypeStruct((N, D), jnp.float32),
        jax.ShapeDtypeStruct((N, D), jnp.float32),
    ],
)


def kernel(x, edge_index, W_mu, b_mu, W_logstd, b_logstd):
    pad_idx = N + (jnp.arange(EP - E, dtype=jnp.int32) % PAD_ROWS)
    src = jnp.concatenate([edge_index[0], pad_idx]).reshape(NS, NBP, B)
    dst = jnp.concatenate([edge_index[1], pad_idx]).reshape(NS, NBP, B)
    ones_up = jnp.ones((B,), jnp.float32)

    degw, sa, sb, _, _ = _sc_kernel(x, src, dst, ones_up)
    degc = degw[:N].reshape(N, 1)
    mu, logstd = _mm(degc, sa, sb, W_mu, W_logstd,
                     b_mu.reshape(1, D), b_logstd.reshape(1, D))
    return (mu, logstd)


# speedup vs baseline: 1.0711x; 1.0711x over previous
import functools

import jax
import jax.numpy as jnp
from jax import lax
from jax.experimental import pallas as pl
from jax.experimental.pallas import tpu as pltpu
from jax.experimental.pallas import tpu_sc as plsc

N = 10000
E = 160000
D = 256
H = 128
NC = 2
NS = 16
B = 80
NBP = 128
PAD_ROWS = 240
PN = N + PAD_ROWS
EP = NS * NBP * B
G = 8
GB2 = 2 * G
NBODY = NBP // GB2
NBUF = 4
DEPTH = 3
B2 = 128
NB2 = NS * NBP * B // (NS * B2)
NBODY2 = NB2 // GB2
ZB = PN // NS
CHUNK = 8
R = 1000
MAGIC = 0x5F3759DF

_mesh = plsc.VectorSubcoreMesh(core_axis_name="c", subcore_axis_name="s")


@functools.partial(
    pl.kernel,
    out_type=(
        jax.ShapeDtypeStruct((PN,), jnp.float32),
        jax.ShapeDtypeStruct((PN, H), jnp.float32),
        jax.ShapeDtypeStruct((PN, H), jnp.float32),
        jax.ShapeDtypeStruct((PN, H), jnp.float32),
        jax.ShapeDtypeStruct((PN, H), jnp.float32),
    ),
    mesh=_mesh,
    scratch_types=[
        pltpu.VMEM((2, G, B), jnp.int32),
        pltpu.VMEM((2, G, B), jnp.int32),
        pltpu.VMEM((2, G, B2), jnp.int32),
        pltpu.VMEM((NBUF, B, H), jnp.float32),
        pltpu.VMEM((B2,), jnp.float32),
        pltpu.VMEM((ZB + 16,), jnp.float32),
        pltpu.VMEM_SHARED((PN,), jnp.float32),
        pltpu.VMEM_SHARED((PN, H), jnp.float32),
        pltpu.SemaphoreType.DMA,
        pltpu.SemaphoreType.DMA,
        pltpu.SemaphoreType.DMA,
        pltpu.SemaphoreType.DMA,
        pltpu.SemaphoreType.DMA,
    ],
)
def _sc_kernel(x_hbm, srcr, dstr, dst2r, ones_hbm,
               deg_out, outa, outb, xta, xtb,
               src_v, dst_v, dst2_v, rows_v, ones_v, degb, dacc, acc,
               gsem, isem, ssem, dsem, xsem):
    c = lax.axis_index("c")
    s = lax.axis_index("s")
    r0 = pl.multiple_of(s * ZB, 8)
    coff = pl.multiple_of(c * H, H)

    def x_chunk(k):
        row0 = pl.multiple_of(r0 + 80 * k, 8)
        return x_hbm.at[pl.ds(row0, 80), pl.ds(coff, H)]

    for k in range(DEPTH):
        pltpu.async_copy(x_chunk(k), rows_v.at[k], xsem)

    for i in range(ZB // 16):
        degb[pl.ds(16 * i, 16)] = jnp.zeros((16,), jnp.float32)
    pltpu.sync_copy(degb.at[pl.ds(0, ZB)], dacc.at[pl.ds(r0, ZB)])
    pltpu.sync_copy(ones_hbm, ones_v)
    plsc.subcore_barrier()

    def dst_load(slot, g):
        off = pl.multiple_of(g * G, G)
        pltpu.async_copy(dst2r.at[s, pl.ds(off, G)], dst2_v.at[slot], isem)

    def dst_wait(slot):
        pltpu.make_async_copy(
            dst2r.at[s, pl.ds(0, G)], dst2_v.at[slot], isem).wait()

    def fire8(slot):
        for b in range(G):
            pltpu.async_copy(ones_v, dacc.at[dst2_v.at[slot, b]], dsem,
                             add=True)

    def drain(k):
        for _ in range(k):
            pltpu.make_async_copy(ones_v, dacc.at[pl.ds(0, B2)], dsem).wait()

    pltpu.sync_copy(dst2r.at[s, pl.ds(0, G)], dst2_v.at[0])

    def pa_body(i, carry):
        @pl.when(i > 0)
        def _():
            drain(G)
            dst_wait(0)
        dst_load(1, 2 * i + 1)
        fire8(0)
        dst_wait(1)
        fire8(1)

        @pl.when(i < NBODY2 - 1)
        def _():
            drain(G)
            dst_load(0, 2 * i + 2)
        return carry

    lax.fori_loop(0, NBODY2, pa_body, 0)
    drain(2 * G)
    plsc.subcore_barrier()

    pltpu.sync_copy(dacc.at[pl.ds(r0, ZB)], degb.at[pl.ds(0, ZB)])

    @pl.when(c == 0)
    def _():
        pltpu.sync_copy(degb.at[pl.ds(0, ZB)], deg_out.at[pl.ds(r0, ZB)])

    for v in range(ZB // 16):
        sl = pl.ds(16 * v, 16)
        d = degb[sl] + 1.0
        y = lax.bitcast_convert_type(
            jnp.int32(MAGIC) - lax.shift_right_logical(
                lax.bitcast_convert_type(d, jnp.int32), 1), jnp.float32)
        for _ in range(3):
            y = y * (1.5 - 0.5 * d * y * y)
        degb[sl] = y

    def run(tab, out):
        for k in range(CHUNK):
            def chunk(k=k):
                buf = k % NBUF
                row0 = pl.multiple_of(r0 + 80 * k, 8)
                pltpu.make_async_copy(x_chunk(k), rows_v.at[buf], xsem).wait()
                nk = k + DEPTH
                if nk < CHUNK:
                    def nxt(nk=nk):
                        pltpu.async_copy(
                            x_chunk(nk), rows_v.at[nk % NBUF], xsem)
                    if nk < 5:
                        nxt()
                    else:
                        pl.when(s < NS - 1)(nxt)

                def row_scale(r, carry):
                    dv = degb[pl.ds(80 * k + r, 16)]
                    yv = jnp.broadcast_to(dv[0], (16,))
                    for j in range(H // 16):
                        sl = pl.ds(16 * j, 16)
                        rows_v[buf, r, sl] = rows_v[buf, r, sl] * yv
                    return carry

                lax.fori_loop(0, 80, row_scale, 0)
                pltpu.sync_copy(rows_v.at[buf], acc.at[pl.ds(row0, 80)])
                pltpu.sync_copy(rows_v.at[buf], tab.at[pl.ds(row0, 80)])

            if k < 5:
                chunk()
            else:
                pl.when(s < NS - 1)(chunk)
        plsc.subcore_barrier()

        def idx_load(slot, g):
            off = pl.multiple_of(g * G, G)
            pltpu.async_copy(srcr.at[s, pl.ds(off, G)], src_v.at[slot], isem)
            pltpu.async_copy(dstr.at[s, pl.ds(off, G)], dst_v.at[slot], isem)

        def idx_wait(slot):
            pltpu.make_async_copy(
                srcr.at[s, pl.ds(0, G)], src_v.at[slot], isem).wait()
            pltpu.make_async_copy(
                dstr.at[s, pl.ds(0, G)], dst_v.at[slot], isem).wait()

        def gather(slot, row, buf):
            pltpu.async_copy(tab.at[src_v.at[slot, row]], rows_v.at[buf], gsem)

        def gather_wait(buf):
            pltpu.make_async_copy(
                tab.at[pl.ds(0, B)], rows_v.at[buf], gsem).wait()

        def scatter_wait(buf):
            pltpu.make_async_copy(
                rows_v.at[buf], acc.at[pl.ds(0, B)], ssem).wait()

        pltpu.sync_copy(srcr.at[s, pl.ds(0, G)], src_v.at[0])
        pltpu.sync_copy(dstr.at[s, pl.ds(0, G)], dst_v.at[0])
        for b in range(DEPTH):
            gather(0, b, b)

        def body(i, carry):
            idx_load(1, 2 * i + 1)
            for t in range(GB2):
                gather_wait(t % NBUF)
                if t == 0:
                    @pl.when(i > 0)
                    def _():
                        scatter_wait((t + 3) % NBUF)
                else:
                    scatter_wait((t + 3) % NBUF)
                nt = t + DEPTH
                if nt < GB2:
                    slot, row = (0, nt) if nt < G else (1, nt - G)
                    if nt == G:
                        idx_wait(1)
                    gather(slot, row, nt % NBUF)
                else:
                    @pl.when(i < NBODY - 1)
                    def _(nt=nt):
                        if nt == GB2:
                            idx_wait(0)
                        gather(0, nt - GB2, nt % NBUF)
                if t == G:
                    @pl.when(i < NBODY - 1)
                    def _():
                        idx_load(0, 2 * i + 2)
                sl, rw = (0, t) if t < G else (1, t - G)
                pltpu.async_copy(
                    rows_v.at[t % NBUF], acc.at[dst_v.at[sl, rw]], ssem,
                    add=True)
            return carry

        lax.fori_loop(0, NBODY, body, 0)
        scatter_wait((NBP - 1) % NBUF)
        plsc.subcore_barrier()
        pltpu.sync_copy(acc.at[pl.ds(r0, ZB)], out.at[pl.ds(r0, ZB)])

    @pl.when(c == 0)
    def _():
        run(xta, outa)

    @pl.when(c == 1)
    def _():
        run(xtb, outb)


def _mm_body(deg_ref, sa_ref, sb_ref, wmu_ref, wls_ref, bmu_ref, bls_ref,
             mu_ref, ls_ref):
    dinv = lax.rsqrt(deg_ref[...] + 1.0)
    p0 = sa_ref[...] * dinv
    p1 = sb_ref[...] * dinv
    wmu = wmu_ref[...]
    wls = wls_ref[...]
    mu_ref[...] = (
        jnp.dot(p0, wmu[:H, :], preferred_element_type=jnp.float32)
        + jnp.dot(p1, wmu[H:, :], preferred_element_type=jnp.float32)
        + bmu_ref[...]
    )
    ls_ref[...] = (
        jnp.dot(p0, wls[:H, :], preferred_element_type=jnp.float32)
        + jnp.dot(p1, wls[H:, :], preferred_element_type=jnp.float32)
        + bls_ref[...]
    )


_mm = pl.pallas_call(
    _mm_body,
    grid=(N // R,),
    in_specs=[
        pl.BlockSpec((R, 1), lambda i: (i, 0)),
        pl.BlockSpec((R, H), lambda i: (i, 0)),
        pl.BlockSpec((R, H), lambda i: (i, 0)),
        pl.BlockSpec((D, D), lambda i: (0, 0)),
        pl.BlockSpec((D, D), lambda i: (0, 0)),
        pl.BlockSpec((1, D), lambda i: (0, 0)),
        pl.BlockSpec((1, D), lambda i: (0, 0)),
    ],
    out_specs=[
        pl.BlockSpec((R, D), lambda i: (i, 0)),
        pl.BlockSpec((R, D), lambda i: (i, 0)),
    ],
    out_shape=[
        jax.ShapeDtypeStruct((N, D), jnp.float32),
        jax.ShapeDtypeStruct((N, D), jnp.float32),
    ],
)


def kernel(x, edge_index, W_mu, b_mu, W_logstd, b_logstd):
    pad_idx = N + (jnp.arange(EP - E, dtype=jnp.int32) % PAD_ROWS)
    src = jnp.concatenate([edge_index[0], pad_idx]).reshape(NS, NBP, B)
    dst_flat = jnp.concatenate([edge_index[1], pad_idx])
    dst = dst_flat.reshape(NS, NBP, B)
    dst2 = dst_flat.reshape(NS, NB2, B2)
    ones_up = jnp.ones((B2,), jnp.float32)

    degw, sa, sb, _, _ = _sc_kernel(x, src, dst, dst2, ones_up)
    degc = degw[:N].reshape(N, 1)
    mu, logstd = _mm(degc, sa, sb, W_mu, W_logstd,
                     b_mu.reshape(1, D), b_logstd.reshape(1, D))
    return (mu, logstd)
